# baseline (speedup 1.0000x reference)
import jax
import jax.numpy as jnp
from jax import lax
from jax.experimental import pallas as pl
from jax.experimental.pallas import tpu as pltpu

_N = 16384
_C = 100
_S = 30.0
_BN = 2048
_NCORE = 2
_NB = _N // (_BN * _NCORE)


def _body(x_ref, t_ref, ml_ref, out_ref):
    i = pl.program_id(1)
    x = x_ref[...]
    t = t_ref[...]
    ml = ml_ref[...]
    col = lax.broadcasted_iota(jnp.int32, (_BN, _C), 1)
    onehot = col == t
    logits = x * _S - jnp.where(onehot, ml * _S, 0.0)
    m = jnp.max(logits, axis=1, keepdims=True)
    se = jnp.sum(jnp.exp(logits - m), axis=1, keepdims=True)
    tgt = jnp.sum(jnp.where(onehot, logits, 0.0), axis=1, keepdims=True)
    part = jnp.sum(m + jnp.log(se) - tgt)

    @pl.when(i == 0)
    def _():
        out_ref[0, 0, 0] = 0.0

    out_ref[0, 0, 0] += part

    @pl.when(i == _NB - 1)
    def _():
        out_ref[0, 0, 0] = out_ref[0, 0, 0] / _N


def kernel(x, target, m_list):
    out = pl.pallas_call(
        _body,
        grid=(_NCORE, _NB),
        in_specs=[
            pl.BlockSpec((_BN, _C), lambda o, i: (o * _NB + i, 0)),
            pl.BlockSpec((_BN, 1), lambda o, i: (o * _NB + i, 0)),
            pl.BlockSpec((1, _C), lambda o, i: (0, 0)),
        ],
        out_specs=pl.BlockSpec(
            (1, 1, 1), lambda o, i: (o, 0, 0), memory_space=pltpu.SMEM
        ),
        out_shape=jax.ShapeDtypeStruct((_NCORE, 1, 1), jnp.float32),
        compiler_params=pltpu.CompilerParams(
            dimension_semantics=("parallel", "arbitrary"),
        ),
    )(x, target.reshape(_N, 1), m_list.reshape(1, _C))
    return jnp.sum(out)


# speedup vs baseline: 1.0576x; 1.0576x over previous
import jax
import jax.numpy as jnp
from jax import lax
from jax.experimental import pallas as pl
from jax.experimental.pallas import tpu as pltpu

_N = 16384
_C = 100
_S = 30.0
_BN = 8192
_NB = _N // _BN


def _body(x_ref, t_ref, ml_ref, out_ref):
    i = pl.program_id(0)
    x = x_ref[...]
    t = t_ref[...]
    ml = ml_ref[...]
    col = lax.broadcasted_iota(jnp.int32, (_BN, _C), 1)
    onehot = col == t
    logits = x * _S - jnp.where(onehot, ml * _S, 0.0)
    m = jnp.max(logits, axis=1, keepdims=True)
    se = jnp.sum(jnp.exp(logits - m), axis=1, keepdims=True)
    tgt = jnp.sum(jnp.where(onehot, logits, 0.0), axis=1, keepdims=True)
    part = jnp.sum(m + jnp.log(se) - tgt)

    @pl.when(i == 0)
    def _():
        out_ref[0, 0] = 0.0

    out_ref[0, 0] += part

    @pl.when(i == _NB - 1)
    def _():
        out_ref[0, 0] = out_ref[0, 0] / _N


def kernel(x, target, m_list):
    out = pl.pallas_call(
        _body,
        grid=(_NB,),
        in_specs=[
            pl.BlockSpec((_BN, _C), lambda i: (i, 0)),
            pl.BlockSpec((_BN, 1), lambda i: (i, 0)),
            pl.BlockSpec((1, _C), lambda i: (0, 0)),
        ],
        out_specs=pl.BlockSpec(memory_space=pltpu.SMEM),
        out_shape=jax.ShapeDtypeStruct((1, 1), jnp.float32),
        compiler_params=pltpu.CompilerParams(
            dimension_semantics=("arbitrary",),
        ),
    )(x, target.reshape(_N, 1), m_list.reshape(1, _C))
    return out[0, 0]


# speedup vs baseline: 1.0920x; 1.0325x over previous
import jax
import jax.numpy as jnp
from jax import lax
from jax.experimental import pallas as pl
from jax.experimental.pallas import tpu as pltpu

_N = 16384
_C = 100
_S = 30.0
_BN = 4096
_NB = _N // _BN


def _body(x_ref, t_ref, ml_ref, out_ref):
    i = pl.program_id(0)
    x = x_ref[...]
    t = t_ref[...]
    ml = ml_ref[...]
    col = lax.broadcasted_iota(jnp.int32, (_BN, _C), 1)
    onehot = col == t
    logits = x * _S - jnp.where(onehot, ml * _S, 0.0)
    m = jnp.max(logits, axis=1, keepdims=True)
    se = jnp.sum(jnp.exp(logits - m), axis=1, keepdims=True)
    tgt = jnp.sum(jnp.where(onehot, logits, 0.0), axis=1, keepdims=True)
    part = jnp.sum(m + jnp.log(se) - tgt)

    @pl.when(i == 0)
    def _():
        out_ref[0, 0] = 0.0

    out_ref[0, 0] += part

    @pl.when(i == _NB - 1)
    def _():
        out_ref[0, 0] = out_ref[0, 0] / _N


def kernel(x, target, m_list):
    out = pl.pallas_call(
        _body,
        grid=(_NB,),
        in_specs=[
            pl.BlockSpec((_BN, _C), lambda i: (i, 0)),
            pl.BlockSpec((_BN, 1), lambda i: (i, 0)),
            pl.BlockSpec((1, _C), lambda i: (0, 0)),
        ],
        out_specs=pl.BlockSpec(memory_space=pltpu.SMEM),
        out_shape=jax.ShapeDtypeStruct((1, 1), jnp.float32),
        compiler_params=pltpu.CompilerParams(
            dimension_semantics=("arbitrary",),
        ),
    )(x, target.reshape(_N, 1), m_list.reshape(1, _C))
    return out[0, 0]


# speedup vs baseline: 1.3757x; 1.2598x over previous
import jax
import jax.numpy as jnp
from jax import lax
from jax.experimental import pallas as pl
from jax.experimental.pallas import tpu as pltpu

_N = 16384
_C = 100
_S = 30.0
_BN = 4096
_NB = _N // _BN
_G = _BN // 128


def _body(x_ref, t_ref, ml_ref, out_ref):
    i = pl.program_id(0)
    x = x_ref[...].reshape(_G, 128, _C)
    t = t_ref[...].reshape(_G, 128, 1)
    ml = ml_ref[...]
    col = lax.broadcasted_iota(jnp.int32, (_G, 128, _C), 2)
    onehot = col == t
    logits = x * _S - jnp.where(onehot, (ml * _S).reshape(1, 1, _C), 0.0)
    m = jnp.max(logits, axis=2, keepdims=True)
    se = jnp.sum(jnp.exp(logits - m), axis=2, keepdims=True)
    tgt = jnp.sum(jnp.where(onehot, logits, 0.0), axis=2, keepdims=True)
    part = jnp.sum(m + jnp.log(se) - tgt)

    @pl.when(i == 0)
    def _():
        out_ref[0, 0] = 0.0

    out_ref[0, 0] += part

    @pl.when(i == _NB - 1)
    def _():
        out_ref[0, 0] = out_ref[0, 0] / _N


def kernel(x, target, m_list):
    out = pl.pallas_call(
        _body,
        grid=(_NB,),
        in_specs=[
            pl.BlockSpec((_BN, _C), lambda i: (i, 0)),
            pl.BlockSpec((_G, 128), lambda i: (i, 0)),
            pl.BlockSpec((1, _C), lambda i: (0, 0)),
        ],
        out_specs=pl.BlockSpec(memory_space=pltpu.SMEM),
        out_shape=jax.ShapeDtypeStruct((1, 1), jnp.float32),
        compiler_params=pltpu.CompilerParams(
            dimension_semantics=("arbitrary",),
        ),
    )(x, target.reshape(_N // 128, 128), m_list.reshape(1, _C))
    return out[0, 0]


# speedup vs baseline: 3.3391x; 2.4272x over previous
import jax
import jax.numpy as jnp
from jax import lax
from jax.experimental import pallas as pl
from jax.experimental.pallas import tpu as pltpu

_N = 16384
_C = 100
_S = 30.0
_BT = 4096
_NB = _N // _BT


def _body(xt_ref, t_ref, ml_ref, out_ref):
    i = pl.program_id(0)
    xt = xt_ref[...]
    t = t_ref[...]
    ml = ml_ref[...]
    row = lax.broadcasted_iota(jnp.int32, (_C, _BT), 0)
    onehot = row == t
    logits = xt * _S - jnp.where(onehot, ml * _S, 0.0)
    m = jnp.max(logits, axis=0, keepdims=True)
    se = jnp.sum(jnp.exp(logits - m), axis=0, keepdims=True)
    tgt = jnp.sum(jnp.where(onehot, logits, 0.0), axis=0, keepdims=True)
    part = jnp.sum(m + jnp.log(se) - tgt)

    @pl.when(i == 0)
    def _():
        out_ref[0, 0] = 0.0

    out_ref[0, 0] += part

    @pl.when(i == _NB - 1)
    def _():
        out_ref[0, 0] = out_ref[0, 0] / _N


def kernel(x, target, m_list):
    out = pl.pallas_call(
        _body,
        grid=(_NB,),
        in_specs=[
            pl.BlockSpec((_C, _BT), lambda i: (0, i)),
            pl.BlockSpec((1, _BT), lambda i: (0, i)),
            pl.BlockSpec((_C, 1), lambda i: (0, 0)),
        ],
        out_specs=pl.BlockSpec(memory_space=pltpu.SMEM),
        out_shape=jax.ShapeDtypeStruct((1, 1), jnp.float32),
        compiler_params=pltpu.CompilerParams(
            dimension_semantics=("arbitrary",),
        ),
    )(x.T, target.reshape(1, _N), m_list.reshape(_C, 1))
    return out[0, 0]
